# initial kernel scaffold (unmeasured)
import jax
import jax.numpy as jnp
from jax import lax
from jax.experimental import pallas as pl
from jax.experimental.pallas import tpu as pltpu

N_DEV = 4


def kernel(x, w_mat):
    m_glob, k_loc = x.shape
    _, n_full = w_mat.shape
    m_chunk = m_glob // N_DEV
    n_half = n_full // 2
    bf16 = jnp.bfloat16
    f32 = jnp.float32

    def body(x_ref, w_ref, out_ref, send_buf, recv_buf, send_sems, recv_sems):
        my = lax.axis_index("i")
        right = (my + 1) % N_DEV
        left = (my + N_DEV - 1) % N_DEV

        barrier_sem = pltpu.get_barrier_semaphore()
        for nbr in (left, right):
            pl.semaphore_signal(
                barrier_sem, inc=1,
                device_id=(nbr,), device_id_type=pl.DeviceIdType.MESH,
            )
        pl.semaphore_wait(barrier_sem, 2)

        def partial(c, d):
            xc = x_ref[pl.ds(c * m_chunk, m_chunk), :]
            wh = w_ref[:, d * n_half:(d + 1) * n_half]
            return jnp.dot(xc, wh, preferred_element_type=f32)

        send_buf[0, 0] = partial((my - 1) % N_DEV, 0).astype(bf16)
        send_buf[1, 0] = partial((my + 1) % N_DEV, 1).astype(bf16)

        slot_rdma = {}
        for s in range(N_DEV - 1):
            slot = s % 2
            rd = []
            for d, tgt in ((0, right), (1, left)):
                r = pltpu.make_async_remote_copy(
                    src_ref=send_buf.at[d, slot],
                    dst_ref=recv_buf.at[d, s],
                    send_sem=send_sems.at[d, slot],
                    recv_sem=recv_sems.at[d, s],
                    device_id=(tgt,),
                    device_id_type=pl.DeviceIdType.MESH,
                )
                r.start()
                rd.append(r)

            nslot = (s + 1) % 2
            if (0, nslot) in slot_rdma:
                slot_rdma[(0, nslot)].wait_send()
                slot_rdma[(1, nslot)].wait_send()
            slot_rdma[(0, slot)] = rd[0]
            slot_rdma[(1, slot)] = rd[1]

            c_cw = (my - 2 - s) % N_DEV
            c_ccw = (my + 2 + s) % N_DEV
            send_buf[0, nslot] = partial(c_cw, 0).astype(bf16)
            send_buf[1, nslot] = partial(c_ccw, 1).astype(bf16)

            rd[0].wait_recv()
            rd[1].wait_recv()

            if s < N_DEV - 2:
                send_buf[0, nslot] = (
                    send_buf[0, nslot].astype(f32) + recv_buf[0, s].astype(f32)
                ).astype(bf16)
                send_buf[1, nslot] = (
                    send_buf[1, nslot].astype(f32) + recv_buf[1, s].astype(f32)
                ).astype(bf16)
            else:
                for d in (0, 1):
                    acc = (
                        send_buf[d, nslot].astype(f32)
                        + recv_buf[d, s].astype(f32)
                    )
                    z = jnp.clip(acc, -60.0, 60.0)
                    out_ref[:, d * n_half:(d + 1) * n_half] = (
                        acc / (1.0 + jnp.exp(-z))
                    )

        slot_rdma[(0, (N_DEV - 2) % 2)].wait_send()
        slot_rdma[(1, (N_DEV - 2) % 2)].wait_send()

    return pl.pallas_call(
        body,
        out_shape=jax.ShapeDtypeStruct((m_chunk, n_full), f32),
        in_specs=[
            pl.BlockSpec(memory_space=pltpu.VMEM),
            pl.BlockSpec(memory_space=pltpu.VMEM),
        ],
        out_specs=pl.BlockSpec(memory_space=pltpu.VMEM),
        scratch_shapes=[
            pltpu.VMEM((2, 2, m_chunk, n_half), bf16),
            pltpu.VMEM((2, N_DEV - 1, m_chunk, n_half), bf16),
            pltpu.SemaphoreType.DMA((2, 2)),
            pltpu.SemaphoreType.DMA((2, N_DEV - 1)),
        ],
        compiler_params=pltpu.CompilerParams(collective_id=0),
    )(x, w_mat)


# baseline (device time: 347381 ns/iter reference)
import jax
import jax.numpy as jnp
from jax import lax
from jax.experimental import pallas as pl
from jax.experimental.pallas import tpu as pltpu

N_DEV = 4
N_HOP = N_DEV - 1
T = 8

bf16 = jnp.bfloat16
f32 = jnp.float32


def kernel(x, w_mat):
    m_glob, k_loc = x.shape
    _, n_full = w_mat.shape
    m_chunk = m_glob // N_DEV
    n_half = n_full // 2
    wt = n_half // T

    xb = x.astype(bf16)
    wb = w_mat.astype(bf16)

    def body(x_ref, w_ref, out_ref, a_buf, r_buf, send_sems, recv_sems,
             credit_sems):
        my = lax.axis_index("i")
        right = (my + 1) % N_DEV
        left = (my + N_DEV - 1) % N_DEV
        tgt = {0: right, 1: left}
        src = {0: left, 1: right}

        barrier_sem = pltpu.get_barrier_semaphore()
        for nbr in (left, right):
            pl.semaphore_signal(
                barrier_sem, inc=1,
                device_id=(nbr,), device_id_type=pl.DeviceIdType.MESH,
            )
        pl.semaphore_wait(barrier_sem, 2)

        def partial_f32(c, d, t):
            xc = x_ref[pl.ds(c * m_chunk, m_chunk), :]
            wc = w_ref[:, d * n_half + t * wt: d * n_half + (t + 1) * wt]
            return jnp.dot(xc, wc, preferred_element_type=f32)

        def chunk(d, s):
            return (my - 1 - s) % N_DEV if d == 0 else (my + 1 + s) % N_DEV

        rdmas = {0: [], 1: []}

        def build_and_send(d, m):
            s, t = divmod(m, T)
            p = partial_f32(chunk(d, s), d, t)
            if m >= T:
                rdmas[d][m - T].wait_recv()
            if m >= 2:
                rdmas[d][m - 2].wait_send()
            if m >= T:
                a_buf[d, m % 2] = (p + r_buf[d, m % T].astype(f32)).astype(bf16)
                pl.semaphore_signal(
                    credit_sems.at[d, m % T], inc=1,
                    device_id=(src[d],), device_id_type=pl.DeviceIdType.MESH,
                )
            else:
                a_buf[d, m % 2] = p.astype(bf16)
            if m >= T:
                pl.semaphore_wait(credit_sems.at[d, m % T], 1)
            r = pltpu.make_async_remote_copy(
                src_ref=a_buf.at[d, m % 2],
                dst_ref=r_buf.at[d, m % T],
                send_sem=send_sems.at[d, m % 2],
                recv_sem=recv_sems.at[d, m % T],
                device_id=(tgt[d],),
                device_id_type=pl.DeviceIdType.MESH,
            )
            r.start()
            rdmas[d].append(r)

        for m in range(N_HOP * T):
            for d in (0, 1):
                build_and_send(d, m)

        for t in range(T):
            for d in (0, 1):
                pm = (N_HOP - 1) * T + t
                p = partial_f32(my, d, t)
                rdmas[d][pm].wait_recv()
                acc = p + r_buf[d, pm % T].astype(f32)
                z = jnp.clip(acc, -60.0, 60.0)
                col = d * n_half + t * wt
                out_ref[:, col:col + wt] = (
                    acc / (1.0 + jnp.exp(-z))
                ).astype(bf16)

        for d in (0, 1):
            rdmas[d][N_HOP * T - 2].wait_send()
            rdmas[d][N_HOP * T - 1].wait_send()

    out = pl.pallas_call(
        body,
        out_shape=jax.ShapeDtypeStruct((m_chunk, n_full), bf16),
        in_specs=[
            pl.BlockSpec(memory_space=pltpu.VMEM),
            pl.BlockSpec(memory_space=pltpu.VMEM),
        ],
        out_specs=pl.BlockSpec(memory_space=pltpu.VMEM),
        scratch_shapes=[
            pltpu.VMEM((2, 2, m_chunk, wt), bf16),
            pltpu.VMEM((2, T, m_chunk, wt), bf16),
            pltpu.SemaphoreType.DMA((2, 2)),
            pltpu.SemaphoreType.DMA((2, T)),
            pltpu.SemaphoreType.REGULAR((2, T)),
        ],
        compiler_params=pltpu.CompilerParams(
            collective_id=0,
            vmem_limit_bytes=66_584_576,
        ),
    )(xb, wb)
    return out.astype(f32)


# device time: 342670 ns/iter; 1.0137x vs baseline; 1.0137x over previous
import jax
import jax.numpy as jnp
from jax import lax
from jax.experimental import pallas as pl
from jax.experimental.pallas import tpu as pltpu

N_DEV = 4
N_HOP = N_DEV - 1
T = 8

bf16 = jnp.bfloat16
f32 = jnp.float32


def kernel(x, w_mat):
    m_glob, k_loc = x.shape
    _, n_full = w_mat.shape
    m_chunk = m_glob // N_DEV
    n_half = n_full // 2
    wt = n_half // T

    xb = x.astype(bf16)
    wb = w_mat.astype(bf16)

    def body(x_ref, w_ref, out_ref, a_buf, r_buf, send_sems, recv_sems,
             credit_sems):
        my = lax.axis_index("i")
        right = (my + 1) % N_DEV
        left = (my + N_DEV - 1) % N_DEV
        tgt = {0: right, 1: left}
        src = {0: left, 1: right}

        barrier_sem = pltpu.get_barrier_semaphore()
        for nbr in (left, right):
            pl.semaphore_signal(
                barrier_sem, inc=1,
                device_id=(nbr,), device_id_type=pl.DeviceIdType.MESH,
            )
        pl.semaphore_wait(barrier_sem, 2)

        def partial_f32(c, d, t):
            xc = x_ref[pl.ds(c * m_chunk, m_chunk), :]
            wc = w_ref[:, d * n_half + t * wt: d * n_half + (t + 1) * wt]
            return jnp.dot(xc, wc, preferred_element_type=f32)

        def chunk(d, s):
            return (my - 1 - s) % N_DEV if d == 0 else (my + 1 + s) % N_DEV

        rdmas = {0: [], 1: []}

        def build_and_send(d, m):
            s, t = divmod(m, T)
            p = partial_f32(chunk(d, s), d, t)
            if m >= T:
                rdmas[d][m - T].wait_recv()
            if m >= 2:
                rdmas[d][m - 2].wait_send()
            if m >= T:
                a_buf[d, m % 2] = (p + r_buf[d, m % T].astype(f32)).astype(bf16)
                pl.semaphore_signal(
                    credit_sems.at[d, m % T], inc=1,
                    device_id=(src[d],), device_id_type=pl.DeviceIdType.MESH,
                )
            else:
                a_buf[d, m % 2] = p.astype(bf16)
            if m >= T:
                pl.semaphore_wait(credit_sems.at[d, m % T], 1)
            r = pltpu.make_async_remote_copy(
                src_ref=a_buf.at[d, m % 2],
                dst_ref=r_buf.at[d, m % T],
                send_sem=send_sems.at[d, m % 2],
                recv_sem=recv_sems.at[d, m % T],
                device_id=(tgt[d],),
                device_id_type=pl.DeviceIdType.MESH,
            )
            r.start()
            rdmas[d].append(r)

        for m in range(N_HOP * T):
            for d in (0, 1):
                build_and_send(d, m)

        for t in range(T):
            for d in (0, 1):
                pm = (N_HOP - 1) * T + t
                p = partial_f32(my, d, t)
                rdmas[d][pm].wait_recv()
                acc = p + r_buf[d, pm % T].astype(f32)
                z = jnp.clip(acc, -60.0, 60.0)
                col = d * n_half + t * wt
                out_ref[:, col:col + wt] = (
                    acc / (1.0 + jnp.exp(-z))
                ).astype(bf16)

        for d in (0, 1):
            rdmas[d][N_HOP * T - 2].wait_send()
            rdmas[d][N_HOP * T - 1].wait_send()

    out = pl.pallas_call(
        body,
        out_shape=jax.ShapeDtypeStruct((m_chunk, n_full), bf16),
        in_specs=[
            pl.BlockSpec(memory_space=pltpu.VMEM),
            pl.BlockSpec(memory_space=pltpu.VMEM),
        ],
        out_specs=pl.BlockSpec(memory_space=pltpu.VMEM),
        scratch_shapes=[
            pltpu.VMEM((2, 2, m_chunk, wt), bf16),
            pltpu.VMEM((2, T, m_chunk, wt), bf16),
            pltpu.SemaphoreType.DMA((2, 2)),
            pltpu.SemaphoreType.DMA((2, T)),
            pltpu.SemaphoreType.REGULAR((2, T)),
        ],
        compiler_params=pltpu.CompilerParams(
            collective_id=0,
            vmem_limit_bytes=66_584_576,
        ),
    )(xb, wb)
    return out


# device time: 328083 ns/iter; 1.0588x vs baseline; 1.0445x over previous
import jax
import jax.numpy as jnp
from jax import lax
from jax.experimental import pallas as pl
from jax.experimental.pallas import tpu as pltpu

N_DEV = 4
N_HOP = N_DEV - 1
T = 8

bf16 = jnp.bfloat16
f32 = jnp.float32


def kernel(x, w_mat):
    m_glob, k_loc = x.shape
    _, n_full = w_mat.shape
    m_chunk = m_glob // N_DEV
    n_half = n_full // 2
    wt = n_half // T
    wp = k_loc
    n_piece = n_full // wp

    xb = x.astype(bf16)

    def body(x_ref, w_ref, out_ref, a_buf, r_buf, wb, wstage, ostage,
             send_sems, recv_sems, credit_sems, wdma_sems, odma_sems):
        my = lax.axis_index("i")
        right = (my + 1) % N_DEV
        left = (my + N_DEV - 1) % N_DEV
        tgt = {0: right, 1: left}
        src = {0: left, 1: right}

        piece_order = []
        for i in range(n_piece // 2):
            piece_order += [i, n_piece // 2 + i]
        wdmas = []

        def start_piece(k):
            j = piece_order[k]
            c = pltpu.make_async_copy(
                w_ref.at[:, j * wp:(j + 1) * wp],
                wstage.at[k % 2],
                wdma_sems.at[k % 2],
            )
            c.start()
            wdmas.append(c)

        def finish_piece(k):
            j = piece_order[k]
            wdmas[k].wait()
            wb[:, j * wp:(j + 1) * wp] = wstage[k % 2].astype(bf16)

        start_piece(0)
        start_piece(1)

        barrier_sem = pltpu.get_barrier_semaphore()
        for nbr in (left, right):
            pl.semaphore_signal(
                barrier_sem, inc=1,
                device_id=(nbr,), device_id_type=pl.DeviceIdType.MESH,
            )
        pl.semaphore_wait(barrier_sem, 2)

        def partial_f32(c, d, t):
            xc = x_ref[pl.ds(c * m_chunk, m_chunk), :]
            wc = wb[:, d * n_half + t * wt: d * n_half + (t + 1) * wt]
            return jnp.dot(xc, wc, preferred_element_type=f32)

        def chunk(d, s):
            return (my - 1 - s) % N_DEV if d == 0 else (my + 1 + s) % N_DEV

        rdmas = {0: [], 1: []}

        def build_and_send(d, m):
            s, t = divmod(m, T)
            p = partial_f32(chunk(d, s), d, t)
            if m >= T:
                rdmas[d][m - T].wait_recv()
            if m >= 2:
                rdmas[d][m - 2].wait_send()
            if m >= T:
                a_buf[d, m % 2] = (p + r_buf[d, m % T].astype(f32)).astype(bf16)
                pl.semaphore_signal(
                    credit_sems.at[d, m % T], inc=1,
                    device_id=(src[d],), device_id_type=pl.DeviceIdType.MESH,
                )
            else:
                a_buf[d, m % 2] = p.astype(bf16)
            if m >= T:
                pl.semaphore_wait(credit_sems.at[d, m % T], 1)
            r = pltpu.make_async_remote_copy(
                src_ref=a_buf.at[d, m % 2],
                dst_ref=r_buf.at[d, m % T],
                send_sem=send_sems.at[d, m % 2],
                recv_sem=recv_sems.at[d, m % T],
                device_id=(tgt[d],),
                device_id_type=pl.DeviceIdType.MESH,
            )
            r.start()
            rdmas[d].append(r)

        for m in range(T):
            if m % 2 == 0:
                for k in (m, m + 1):
                    finish_piece(k)
                    if k + 2 < n_piece:
                        start_piece(k + 2)
            for d in (0, 1):
                build_and_send(d, m)
        for m in range(T, N_HOP * T):
            for d in (0, 1):
                build_and_send(d, m)

        odmas = []
        for t in range(T):
            for d in (0, 1):
                i = len(odmas)
                slot = i % 2
                pm = (N_HOP - 1) * T + t
                p = partial_f32(my, d, t)
                rdmas[d][pm].wait_recv()
                acc = p + r_buf[d, pm % T].astype(f32)
                z = jnp.clip(acc, -60.0, 60.0)
                if i >= 2:
                    odmas[i - 2].wait()
                ostage[slot] = acc / (1.0 + jnp.exp(-z))
                col = d * n_half + t * wt
                c = pltpu.make_async_copy(
                    ostage.at[slot],
                    out_ref.at[:, pl.ds(col, wt)],
                    odma_sems.at[slot],
                )
                c.start()
                odmas.append(c)

        odmas[-2].wait()
        odmas[-1].wait()
        for d in (0, 1):
            rdmas[d][N_HOP * T - 2].wait_send()
            rdmas[d][N_HOP * T - 1].wait_send()

    out = pl.pallas_call(
        body,
        out_shape=jax.ShapeDtypeStruct((m_chunk, n_full), f32),
        in_specs=[
            pl.BlockSpec(memory_space=pltpu.VMEM),
            pl.BlockSpec(memory_space=pl.ANY),
        ],
        out_specs=pl.BlockSpec(memory_space=pl.ANY),
        scratch_shapes=[
            pltpu.VMEM((2, 2, m_chunk, wt), bf16),
            pltpu.VMEM((2, T, m_chunk, wt), bf16),
            pltpu.VMEM((k_loc, n_full), bf16),
            pltpu.VMEM((2, k_loc, wp), f32),
            pltpu.VMEM((2, m_chunk, wt), f32),
            pltpu.SemaphoreType.DMA((2, 2)),
            pltpu.SemaphoreType.DMA((2, T)),
            pltpu.SemaphoreType.REGULAR((2, T)),
            pltpu.SemaphoreType.DMA((2,)),
            pltpu.SemaphoreType.DMA((2,)),
        ],
        compiler_params=pltpu.CompilerParams(
            collective_id=0,
            vmem_limit_bytes=66_584_576,
        ),
    )(xb, w_mat)
    return out


# device time: 324386 ns/iter; 1.0709x vs baseline; 1.0114x over previous
import jax
import jax.numpy as jnp
from jax import lax
from jax.experimental import pallas as pl
from jax.experimental.pallas import tpu as pltpu

N_DEV = 4
N_HOP = N_DEV - 1
T = 8
NR = 10

bf16 = jnp.bfloat16
f32 = jnp.float32


def kernel(x, w_mat):
    m_glob, k_loc = x.shape
    _, n_full = w_mat.shape
    m_chunk = m_glob // N_DEV
    n_half = n_full // 2
    wt = n_half // T
    wp = wt
    n_piece = n_full // wp

    xb = x.astype(bf16)

    def body(x_ref, w_ref, out_ref, a_buf, r_buf, wb, wstage, ostage,
             send_sems, recv_sems, credit_sems, wdma_sems, odma_sems):
        my = lax.axis_index("i")
        right = (my + 1) % N_DEV
        left = (my + N_DEV - 1) % N_DEV
        tgt = {0: right, 1: left}
        src = {0: left, 1: right}

        piece_order = []
        for i in range(n_piece // 2):
            piece_order += [i, n_piece // 2 + i]
        wdmas = []

        def start_piece(k):
            j = piece_order[k]
            c = pltpu.make_async_copy(
                w_ref.at[:, j * wp:(j + 1) * wp],
                wstage.at[k % 2],
                wdma_sems.at[k % 2],
            )
            c.start()
            wdmas.append(c)

        def finish_piece(k):
            j = piece_order[k]
            wdmas[k].wait()
            wb[:, j * wp:(j + 1) * wp] = wstage[k % 2].astype(bf16)

        start_piece(0)
        start_piece(1)

        barrier_sem = pltpu.get_barrier_semaphore()
        for nbr in (left, right):
            pl.semaphore_signal(
                barrier_sem, inc=1,
                device_id=(nbr,), device_id_type=pl.DeviceIdType.MESH,
            )
        pl.semaphore_wait(barrier_sem, 2)

        def partial_f32(c, d, t):
            xc = x_ref[pl.ds(c * m_chunk, m_chunk), :]
            wc = wb[:, d * n_half + t * wt: d * n_half + (t + 1) * wt]
            return jnp.dot(xc, wc, preferred_element_type=f32)

        def chunk(d, s):
            return (my - 1 - s) % N_DEV if d == 0 else (my + 1 + s) % N_DEV

        rdmas = {0: [], 1: []}

        def build_and_send(d, m):
            s, t = divmod(m, T)
            p = partial_f32(chunk(d, s), d, t)
            if m >= T:
                rdmas[d][m - T].wait_recv()
            if m >= 2:
                rdmas[d][m - 2].wait_send()
            if m >= T:
                a_buf[d, m % 2] = (
                    p + r_buf[d, (m - T) % NR].astype(f32)
                ).astype(bf16)
                if m - T + NR < N_HOP * T:
                    pl.semaphore_signal(
                        credit_sems.at[d, (m - T) % NR], inc=1,
                        device_id=(src[d],), device_id_type=pl.DeviceIdType.MESH,
                    )
            else:
                a_buf[d, m % 2] = p.astype(bf16)
            if m >= NR:
                pl.semaphore_wait(credit_sems.at[d, m % NR], 1)
            r = pltpu.make_async_remote_copy(
                src_ref=a_buf.at[d, m % 2],
                dst_ref=r_buf.at[d, m % NR],
                send_sem=send_sems.at[d, m % 2],
                recv_sem=recv_sems.at[d, m % NR],
                device_id=(tgt[d],),
                device_id_type=pl.DeviceIdType.MESH,
            )
            r.start()
            rdmas[d].append(r)

        for m in range(T):
            for k in (2 * m, 2 * m + 1):
                finish_piece(k)
                if k + 2 < n_piece:
                    start_piece(k + 2)
            for d in (0, 1):
                build_and_send(d, m)
        for m in range(T, N_HOP * T):
            for d in (0, 1):
                build_and_send(d, m)

        odmas = []
        for t in range(T):
            for d in (0, 1):
                i = len(odmas)
                slot = i % 2
                pm = (N_HOP - 1) * T + t
                p = partial_f32(my, d, t)
                rdmas[d][pm].wait_recv()
                acc = p + r_buf[d, pm % NR].astype(f32)
                z = jnp.clip(acc, -60.0, 60.0)
                if i >= 2:
                    odmas[i - 2].wait()
                ostage[slot] = acc / (1.0 + jnp.exp(-z))
                col = d * n_half + t * wt
                c = pltpu.make_async_copy(
                    ostage.at[slot],
                    out_ref.at[:, pl.ds(col, wt)],
                    odma_sems.at[slot],
                )
                c.start()
                odmas.append(c)

        odmas[-2].wait()
        odmas[-1].wait()
        for d in (0, 1):
            rdmas[d][N_HOP * T - 2].wait_send()
            rdmas[d][N_HOP * T - 1].wait_send()

    out = pl.pallas_call(
        body,
        out_shape=jax.ShapeDtypeStruct((m_chunk, n_full), f32),
        in_specs=[
            pl.BlockSpec(memory_space=pltpu.VMEM),
            pl.BlockSpec(memory_space=pl.ANY),
        ],
        out_specs=pl.BlockSpec(memory_space=pl.ANY),
        scratch_shapes=[
            pltpu.VMEM((2, 2, m_chunk, wt), bf16),
            pltpu.VMEM((2, NR, m_chunk, wt), bf16),
            pltpu.VMEM((k_loc, n_full), bf16),
            pltpu.VMEM((2, k_loc, wp), f32),
            pltpu.VMEM((2, m_chunk, wt), f32),
            pltpu.SemaphoreType.DMA((2, 2)),
            pltpu.SemaphoreType.DMA((2, NR)),
            pltpu.SemaphoreType.REGULAR((2, NR)),
            pltpu.SemaphoreType.DMA((2,)),
            pltpu.SemaphoreType.DMA((2,)),
        ],
        compiler_params=pltpu.CompilerParams(
            collective_id=0,
            vmem_limit_bytes=66_584_576,
        ),
    )(xb, w_mat)
    return out
